# baseline (speedup 1.0000x reference)
import functools

import jax
import jax.numpy as jnp
from jax import lax
from jax.experimental import pallas as pl
from jax.experimental.pallas import tpu as pltpu
from jax.experimental.pallas import tpu_sc as plsc

H = 1024
F = 2048
E = 8
T = 2048

BM = 128
NT = 40
NMETA = 48
ROWS = NT * BM
NSUB = 16
TPT = T // NSUB
TPW = T // 32

_MESH = plsc.VectorSubcoreMesh(core_axis_name="c", subcore_axis_name="s")


def _router_body(x_ref, wr_ref, gate_ref):
    x = x_ref[...]
    wr = wr_ref[...]
    logits = lax.dot_general(
        x, wr, (((1,), (1,)), ((), ())), preferred_element_type=jnp.float32
    )
    m = jnp.max(logits, axis=-1, keepdims=True)
    p = jnp.exp(logits - m)
    w = p / jnp.sum(p, axis=-1, keepdims=True)
    eidx = lax.broadcasted_iota(jnp.int32, w.shape, 1)
    w1 = jnp.max(w, axis=-1, keepdims=True)
    first1 = jnp.min(jnp.where(w == w1, eidx, E), axis=-1, keepdims=True)
    m1 = eidx == first1
    wm = jnp.where(m1, -jnp.inf, w)
    w2 = jnp.max(wm, axis=-1, keepdims=True)
    first2 = jnp.min(jnp.where(wm == w2, eidx, E), axis=-1, keepdims=True)
    m2 = eidx == first2
    s = w1 + w2
    gate_ref[...] = jnp.where(m1, w1 / s, jnp.where(m2, w2 / s, 0.0))


def _router(xf, W_router):
    BMR = 1024
    return pl.pallas_call(
        _router_body,
        grid=(T // BMR,),
        in_specs=[
            pl.BlockSpec((BMR, H), lambda m: (m, 0)),
            pl.BlockSpec((E, H), lambda m: (0, 0)),
        ],
        out_specs=pl.BlockSpec((BMR, E), lambda m: (m, 0)),
        out_shape=jax.ShapeDtypeStruct((T, E), jnp.float32),
    )(xf, W_router)


def _splat(x):
    return jnp.full((16,), x, jnp.int32)


def _scan16(x, buf):
    for sh in (1, 2, 4, 8):
        buf[pl.ds(16, 16)] = x
        x = x + buf[pl.ds(16 - sh, 16)]
    return x


def _sort_body(gate_hbm, x_hbm, meta_hbm, pos0_hbm, pos1_hbm, ws_hbm,
               xs_hbm, gate_v, e0_v, e1_v, w0_v, w1_v, d0_v, d1_v,
               cntrow_v, cnt_sh, cnt_v, meta_v, scan_v, xrow_v, xrow2_v,
               sem, sem2, wsem):
    c = lax.axis_index("c")
    s = lax.axis_index("s")
    t0 = s * TPT
    lane = lax.broadcasted_iota(jnp.int32, (16,), 0)
    zv = jnp.zeros((16,), jnp.int32)
    ones = zv + 1
    scan_v[pl.ds(0, 16)] = zv

    _sA = jax.named_scope("sort_gatecopy")
    _sA.__enter__()
    pltpu.sync_copy(gate_hbm.at[:, pl.ds(t0, TPT)], gate_v)
    _sA.__exit__(None, None, None)
    _sB = jax.named_scope("sort_AD")
    _sB.__enter__()
    cnt = [zv for _ in range(E)]
    for g in range(TPT // 16):
        e0 = _splat(0)
        e1 = _splat(0)
        w0 = jnp.zeros((16,), jnp.float32)
        w1 = jnp.zeros((16,), jnp.float32)
        nseen = zv
        for e in range(E):
            ge = gate_v[e, pl.ds(g * 16, 16)]
            nz = ge > jnp.zeros((16,), jnp.float32)
            take0 = nz & (nseen == _splat(0))
            take1 = nz & (nseen == _splat(1))
            e0 = jnp.where(take0, _splat(e), e0)
            w0 = jnp.where(take0, ge, w0)
            e1 = jnp.where(take1, _splat(e), e1)
            w1 = jnp.where(take1, ge, w1)
            nseen = nseen + jnp.where(nz, ones, zv)
            cnt[e] = cnt[e] + jnp.where(take0, ones, zv) + jnp.where(take1, ones, zv)
        e0_v[pl.ds(g * 16, 16)] = e0
        e1_v[pl.ds(g * 16, 16)] = e1
        w0_v[pl.ds(g * 16, 16)] = w0
        w1_v[pl.ds(g * 16, 16)] = w1

    crow = zv
    for e in range(E):
        tot = _scan16(cnt[e], scan_v)[15]
        crow = jnp.where(lane == _splat(e), zv + tot, crow)
    cntrow_v[...] = crow
    pltpu.sync_copy(cntrow_v, cnt_sh.at[pl.ds(s * 16, 16)])
    plsc.subcore_barrier()

    pltpu.sync_copy(cnt_sh, cnt_v)
    rows = [cnt_v[pl.ds(t * 16, 16)] for t in range(NSUB)]
    svec = zv + s
    total_vec = rows[0]
    prefix_vec = rows[0] * jnp.where(_splat(0) < svec, ones, zv)
    for t in range(1, NSUB):
        total_vec = total_vec + rows[t]
        prefix_vec = prefix_vec + rows[t] * jnp.where(_splat(t) < svec, ones, zv)
    base = []
    cb = [jnp.int32(0)]
    for e in range(E):
        total = total_vec[e]
        nblk = (total + BM - 1) // BM
        base.append(cb[e] * BM + prefix_vec[e])
        cb.append(cb[e] + nblk)
    n_active = cb[E]

    run = [jnp.int32(0) for _ in range(E)]
    for g in range(TPT // 16):
        e0 = e0_v[pl.ds(g * 16, 16)]
        e1 = e1_v[pl.ds(g * 16, 16)]
        d0 = zv
        d1 = zv
        for e in range(E):
            m0 = e0 == _splat(e)
            mi0 = jnp.where(m0, ones, zv)
            incl0 = _scan16(mi0, scan_v)
            d0 = d0 + mi0 * ((zv + (base[e] + run[e])) + (incl0 - mi0))
            run[e] = run[e] + incl0[15]
            m1 = e1 == _splat(e)
            mi1 = jnp.where(m1, ones, zv)
            incl1 = _scan16(mi1, scan_v)
            d1 = d1 + mi1 * ((zv + (base[e] + run[e])) + (incl1 - mi1))
            run[e] = run[e] + incl1[15]
        d0_v[g // 2, pl.ds((g % 2) * 16, 16)] = d0
        d1_v[g // 2, pl.ds((g % 2) * 16, 16)] = d1
    _sB.__exit__(None, None, None)

    @pl.when(c == 0)
    def _():
        pltpu.sync_copy(d0_v, pos0_hbm.at[s])
        pltpu.sync_copy(d1_v, pos1_hbm.at[s])

    @pl.when((c == 0) & (s == 0))
    def _():
        nav = zv + n_active
        lastev = zv
        for e in range(E):
            lastev = lastev + jnp.where(nav - 1 >= (zv + cb[e + 1]), ones, zv)
        for chk in range(NMETA // 16):
            ev = zv
            for e in range(E):
                cbv = zv + cb[e + 1]
                ev = ev + jnp.where(lane + chk * 16 >= cbv, ones, zv)
            ev = jnp.minimum(ev, lastev)
            ind = jnp.where(lane + chk * 16 == _splat(NT), ones, zv)
            ev = ev + ind * (nav - ev)
            meta_v[pl.ds(chk * 16, 16)] = ev
        pltpu.sync_copy(meta_v, meta_hbm)

    def _scatter(d_v, w_v):
        nch = TPT // 32
        bufs = (xrow_v, xrow2_v)
        loads = [None] * nch
        stores = [None] * nch
        loads[0] = pltpu.async_copy(x_hbm.at[pl.ds(t0, 32)], bufs[0], sem)
        loads[1] = pltpu.async_copy(x_hbm.at[pl.ds(t0 + 32, 32)], bufs[1], sem2)
        wcopies = []
        for ch in range(nch):
            b = ch % 2
            loads[ch].wait()
            stores[ch] = pltpu.async_copy(bufs[b], xs_hbm.at[d_v.at[ch]],
                                          sem if b == 0 else sem2)
            wcopies.append(pltpu.async_copy(w_v.at[pl.ds(ch * 32, 32)],
                                            ws_hbm.at[d_v.at[ch]], wsem))
            if ch + 2 < nch:
                stores[ch].wait()
                loads[ch + 2] = pltpu.async_copy(
                    x_hbm.at[pl.ds(t0 + (ch + 2) * 32, 32)], bufs[b],
                    sem if b == 0 else sem2)
        stores[nch - 2].wait()
        stores[nch - 1].wait()
        for h in wcopies:
            h.wait()

    _sE = jax.named_scope("sort_E")
    _sE.__enter__()

    @pl.when(c == 0)
    def _():
        _scatter(d0_v, w0_v)

    @pl.when(c == 1)
    def _():
        _scatter(d1_v, w1_v)
    _sE.__exit__(None, None, None)


def _sort_scatter(gate, xf):
    f = pl.kernel(
        _sort_body,
        out_type=[
            jax.ShapeDtypeStruct((NMETA,), jnp.int32),
            jax.ShapeDtypeStruct((NSUB, TPT // 32, 32), jnp.int32),
            jax.ShapeDtypeStruct((NSUB, TPT // 32, 32), jnp.int32),
            jax.ShapeDtypeStruct((ROWS,), jnp.float32),
            jax.ShapeDtypeStruct((ROWS, H), jnp.float32),
        ],
        mesh=_MESH,
        scratch_types=[
            pltpu.VMEM((E, TPT), jnp.float32),
            pltpu.VMEM((TPT,), jnp.int32),
            pltpu.VMEM((TPT,), jnp.int32),
            pltpu.VMEM((TPT,), jnp.float32),
            pltpu.VMEM((TPT,), jnp.float32),
            pltpu.VMEM((TPT // 32, 32), jnp.int32),
            pltpu.VMEM((TPT // 32, 32), jnp.int32),
            pltpu.VMEM((16,), jnp.int32),
            pltpu.VMEM_SHARED((NSUB * 16,), jnp.int32),
            pltpu.VMEM((NSUB * 16,), jnp.int32),
            pltpu.VMEM((NMETA,), jnp.int32),
            pltpu.VMEM((32,), jnp.int32),
            pltpu.VMEM((32, H), jnp.float32),
            pltpu.VMEM((32, H), jnp.float32),
            pltpu.SemaphoreType.DMA,
            pltpu.SemaphoreType.DMA,
            pltpu.SemaphoreType.DMA,
        ],
    )
    return f(gate, xf)


def _glu_body(meta_ref, x_ref, w1_ref, v1_ref, w2_ref, ws_ref, out_ref):
    t = pl.program_id(0)

    @pl.when(t < meta_ref[NT])
    def _():
        x = x_ref[...].astype(jnp.bfloat16)
        w1 = w1_ref[0].astype(jnp.bfloat16)
        v1 = v1_ref[0].astype(jnp.bfloat16)
        w2 = w2_ref[0].astype(jnp.bfloat16)
        gp = lax.dot_general(
            x, w1, (((1,), (1,)), ((), ())), preferred_element_type=jnp.float32
        )
        up = lax.dot_general(
            x, v1, (((1,), (1,)), ((), ())), preferred_element_type=jnp.float32
        )
        inter = (gp * lax.logistic(gp)) * up
        dn = lax.dot_general(
            inter.astype(jnp.bfloat16), w2,
            (((1,), (0,)), ((), ())), preferred_element_type=jnp.float32
        )
        out_ref[...] = dn * ws_ref[0]


def _glu_grouped(xs, w1b, v1b, w2b, ws3d, meta):
    grid_spec = pltpu.PrefetchScalarGridSpec(
        num_scalar_prefetch=1,
        grid=(NT,),
        in_specs=[
            pl.BlockSpec((BM, H), lambda t, m: (jnp.minimum(t, m[NT] - 1), 0)),
            pl.BlockSpec((1, F, H), lambda t, m: (m[t], 0, 0)),
            pl.BlockSpec((1, F, H), lambda t, m: (m[t], 0, 0)),
            pl.BlockSpec((1, F, H), lambda t, m: (m[t], 0, 0)),
            pl.BlockSpec((1, BM, 1), lambda t, m: (t, 0, 0)),
        ],
        out_specs=pl.BlockSpec((BM, H), lambda t, m: (t, 0)),
    )
    return pl.pallas_call(
        _glu_body,
        grid_spec=grid_spec,
        out_shape=jax.ShapeDtypeStruct((ROWS, H), jnp.float32),
        compiler_params=pltpu.CompilerParams(vmem_limit_bytes=58 * 1024 * 1024),
    )(meta, xs, w1b, v1b, w2b, ws3d)


def _combine_body(xo_hbm, pos0_hbm, pos1_hbm, out_hbm,
                  p0_v, p1_v, buf0, buf1, obuf, sem):
    c = lax.axis_index("c")
    s = lax.axis_index("s")
    wid = s * 2 + c
    t0 = wid * TPW

    for ch in range(TPW // 16):
        pltpu.sync_copy(pos0_hbm.at[pl.ds(t0 + ch * 16, 16)], p0_v.at[ch])
        pltpu.sync_copy(pos1_hbm.at[pl.ds(t0 + ch * 16, 16)], p1_v.at[ch])
        pltpu.async_copy(xo_hbm.at[p0_v.at[ch]], buf0, sem).wait()
        pltpu.async_copy(xo_hbm.at[p1_v.at[ch]], buf1, sem).wait()

        def _row(r, _):
            for col in range(H // 16):
                sl = pl.ds(col * 16, 16)
                obuf[r, sl] = buf0[r, sl] + buf1[r, sl]
            return _

        lax.fori_loop(0, 16, _row, 0)
        pltpu.sync_copy(obuf, out_hbm.at[pl.ds(t0 + ch * 16, 16)])


def _combine(xo, pos0, pos1):
    f = pl.kernel(
        _combine_body,
        out_type=jax.ShapeDtypeStruct((T, H), jnp.float32),
        mesh=_MESH,
        scratch_types=[
            pltpu.VMEM((TPW // 16, 16), jnp.int32),
            pltpu.VMEM((TPW // 16, 16), jnp.int32),
            pltpu.VMEM((16, H), jnp.float32),
            pltpu.VMEM((16, H), jnp.float32),
            pltpu.VMEM((16, H), jnp.float32),
            pltpu.SemaphoreType.DMA,
        ],
    )
    return f(xo, pos0, pos1)


def kernel(x, W_router, w1, v1, w2):
    bsz, q_len, hidden = x.shape
    xf = x.reshape(-1, hidden)
    gate = _router(xf, W_router)
    meta, pos0, pos1, ws, xs = _sort_scatter(gate.T, xf)
    xo = _glu_grouped(xs, w1, v1, w2, ws.reshape(NT, BM, 1), meta)
    out = _combine(xo, pos0.reshape(T), pos1.reshape(T))
    return out.reshape(bsz, q_len, hidden)


# speedup vs baseline: 1.2447x; 1.2447x over previous
import functools

import jax
import jax.numpy as jnp
from jax import lax
from jax.experimental import pallas as pl
from jax.experimental.pallas import tpu as pltpu
from jax.experimental.pallas import tpu_sc as plsc

H = 1024
F = 2048
E = 8
T = 2048

BM = 256
NT = 24
ROWS = NT * BM
NSUB = 16
TPT = T // NSUB
TPW = T // 32

_MESH = plsc.VectorSubcoreMesh(core_axis_name="c", subcore_axis_name="s")


def _router_body(x_ref, wr_ref, gate_ref):
    x = x_ref[...]
    wr = wr_ref[...]
    logits = lax.dot_general(
        x, wr, (((1,), (1,)), ((), ())), preferred_element_type=jnp.float32
    )
    m = jnp.max(logits, axis=-1, keepdims=True)
    p = jnp.exp(logits - m)
    w = p / jnp.sum(p, axis=-1, keepdims=True)
    eidx = lax.broadcasted_iota(jnp.int32, w.shape, 1)
    w1 = jnp.max(w, axis=-1, keepdims=True)
    first1 = jnp.min(jnp.where(w == w1, eidx, E), axis=-1, keepdims=True)
    m1 = eidx == first1
    wm = jnp.where(m1, -jnp.inf, w)
    w2 = jnp.max(wm, axis=-1, keepdims=True)
    first2 = jnp.min(jnp.where(wm == w2, eidx, E), axis=-1, keepdims=True)
    m2 = eidx == first2
    s = w1 + w2
    gate = jnp.where(m1, w1 / s, jnp.where(m2, w2 / s, 0.0))
    gate_ref[...] = gate.T


def _router(xf, W_router):
    BMR = 1024
    return pl.pallas_call(
        _router_body,
        grid=(T // BMR,),
        in_specs=[
            pl.BlockSpec((BMR, H), lambda m: (m, 0)),
            pl.BlockSpec((E, H), lambda m: (0, 0)),
        ],
        out_specs=pl.BlockSpec((E, BMR), lambda m: (0, m)),
        out_shape=jax.ShapeDtypeStruct((E, T), jnp.float32),
    )(xf, W_router)


def _splat(x):
    return jnp.full((16,), x, jnp.int32)


def _scan16(x, buf):
    for sh in (1, 2, 4, 8):
        buf[pl.ds(16, 16)] = x
        x = x + buf[pl.ds(16 - sh, 16)]
    return x


def _sort_body(gate_hbm, x_hbm, meta_hbm, pos0_hbm, pos1_hbm, ws_hbm,
               xs_hbm, gate_v, e0_v, e1_v, w0_v, w1_v, d0_v, d1_v,
               cntrow_v, cnt_sh, cnt_v, meta_v, scan_v, xrow_v, xrow2_v,
               sem, sem2, wsem):
    c = lax.axis_index("c")
    s = lax.axis_index("s")
    t0 = s * TPT
    lane = lax.broadcasted_iota(jnp.int32, (16,), 0)
    zv = jnp.zeros((16,), jnp.int32)
    ones = zv + 1
    scan_v[pl.ds(0, 16)] = zv

    pltpu.sync_copy(gate_hbm.at[:, pl.ds(t0, TPT)], gate_v)
    cnt = [zv for _ in range(E)]
    for g in range(TPT // 16):
        e0 = _splat(0)
        e1 = _splat(0)
        w0 = jnp.zeros((16,), jnp.float32)
        w1 = jnp.zeros((16,), jnp.float32)
        nseen = zv
        for e in range(E):
            ge = gate_v[e, pl.ds(g * 16, 16)]
            nz = ge > jnp.zeros((16,), jnp.float32)
            take0 = nz & (nseen == _splat(0))
            take1 = nz & (nseen == _splat(1))
            e0 = jnp.where(take0, _splat(e), e0)
            w0 = jnp.where(take0, ge, w0)
            e1 = jnp.where(take1, _splat(e), e1)
            w1 = jnp.where(take1, ge, w1)
            nseen = nseen + jnp.where(nz, ones, zv)
            cnt[e] = cnt[e] + jnp.where(take0, ones, zv) + jnp.where(take1, ones, zv)
        e0_v[pl.ds(g * 16, 16)] = e0
        e1_v[pl.ds(g * 16, 16)] = e1
        w0_v[pl.ds(g * 16, 16)] = w0
        w1_v[pl.ds(g * 16, 16)] = w1

    crow = zv
    for e in range(E):
        tot = _scan16(cnt[e], scan_v)[15]
        crow = jnp.where(lane == _splat(e), zv + tot, crow)
    cntrow_v[...] = crow
    pltpu.sync_copy(cntrow_v, cnt_sh.at[pl.ds(s * 16, 16)])
    plsc.subcore_barrier()

    pltpu.sync_copy(cnt_sh, cnt_v)
    rows = [cnt_v[pl.ds(t * 16, 16)] for t in range(NSUB)]
    svec = zv + s
    total_vec = rows[0]
    prefix_vec = rows[0] * jnp.where(_splat(0) < svec, ones, zv)
    for t in range(1, NSUB):
        total_vec = total_vec + rows[t]
        prefix_vec = prefix_vec + rows[t] * jnp.where(_splat(t) < svec, ones, zv)
    base = []
    cb = [jnp.int32(0)]
    for e in range(E):
        total = total_vec[e]
        nblk = (total + BM - 1) // BM
        base.append(cb[e] * BM + prefix_vec[e])
        cb.append(cb[e] + nblk)
    n_active = cb[E]

    run = [jnp.int32(0) for _ in range(E)]
    for g in range(TPT // 16):
        e0 = e0_v[pl.ds(g * 16, 16)]
        e1 = e1_v[pl.ds(g * 16, 16)]
        d0 = zv
        d1 = zv
        for e in range(E):
            m0 = e0 == _splat(e)
            mi0 = jnp.where(m0, ones, zv)
            incl0 = _scan16(mi0, scan_v)
            d0 = d0 + mi0 * ((zv + (base[e] + run[e])) + (incl0 - mi0))
            run[e] = run[e] + incl0[15]
            m1 = e1 == _splat(e)
            mi1 = jnp.where(m1, ones, zv)
            incl1 = _scan16(mi1, scan_v)
            d1 = d1 + mi1 * ((zv + (base[e] + run[e])) + (incl1 - mi1))
            run[e] = run[e] + incl1[15]
        d0_v[g // 2, pl.ds((g % 2) * 16, 16)] = d0
        d1_v[g // 2, pl.ds((g % 2) * 16, 16)] = d1

    @pl.when(c == 0)
    def _():
        pltpu.sync_copy(d0_v, pos0_hbm.at[s])
        pltpu.sync_copy(d1_v, pos1_hbm.at[s])

    @pl.when((c == 0) & (s == 0))
    def _():
        nav = zv + n_active
        lastev = zv
        ev0 = zv
        ev1 = zv
        for e in range(E):
            cbv = zv + cb[e + 1]
            lastev = lastev + jnp.where(nav - 1 >= cbv, ones, zv)
            ev0 = ev0 + jnp.where(lane >= cbv, ones, zv)
            ev1 = ev1 + jnp.where(lane + 16 >= cbv, ones, zv)
        ev0 = jnp.minimum(ev0, lastev)
        ev1 = jnp.minimum(ev1, lastev)
        ind = jnp.where(lane == _splat(NT - 16), ones, zv)
        ev1 = ev1 + ind * (nav - ev1)
        meta_v[pl.ds(0, 16)] = ev0
        meta_v[pl.ds(16, 16)] = ev1
        pltpu.sync_copy(meta_v, meta_hbm)

    def _scatter(d_v, w_v):
        nch = TPT // 32
        bufs = (xrow_v, xrow2_v)
        loads = [None] * nch
        stores = [None] * nch
        loads[0] = pltpu.async_copy(x_hbm.at[pl.ds(t0, 32)], bufs[0], sem)
        loads[1] = pltpu.async_copy(x_hbm.at[pl.ds(t0 + 32, 32)], bufs[1], sem2)
        wcopies = []
        for ch in range(nch):
            b = ch % 2
            loads[ch].wait()
            stores[ch] = pltpu.async_copy(bufs[b], xs_hbm.at[d_v.at[ch]],
                                          sem if b == 0 else sem2)
            wcopies.append(pltpu.async_copy(w_v.at[pl.ds(ch * 32, 32)],
                                            ws_hbm.at[d_v.at[ch]], wsem))
            if ch + 2 < nch:
                stores[ch].wait()
                loads[ch + 2] = pltpu.async_copy(
                    x_hbm.at[pl.ds(t0 + (ch + 2) * 32, 32)], bufs[b],
                    sem if b == 0 else sem2)
        stores[nch - 2].wait()
        stores[nch - 1].wait()
        for h in wcopies:
            h.wait()

    @pl.when(c == 0)
    def _():
        _scatter(d0_v, w0_v)

    @pl.when(c == 1)
    def _():
        _scatter(d1_v, w1_v)


def _sort_scatter(gate, xf):
    f = pl.kernel(
        _sort_body,
        out_type=[
            jax.ShapeDtypeStruct((32,), jnp.int32),
            jax.ShapeDtypeStruct((NSUB, TPT // 32, 32), jnp.int32),
            jax.ShapeDtypeStruct((NSUB, TPT // 32, 32), jnp.int32),
            jax.ShapeDtypeStruct((ROWS,), jnp.float32),
            jax.ShapeDtypeStruct((ROWS, H), jnp.float32),
        ],
        mesh=_MESH,
        scratch_types=[
            pltpu.VMEM((E, TPT), jnp.float32),
            pltpu.VMEM((TPT,), jnp.int32),
            pltpu.VMEM((TPT,), jnp.int32),
            pltpu.VMEM((TPT,), jnp.float32),
            pltpu.VMEM((TPT,), jnp.float32),
            pltpu.VMEM((TPT // 32, 32), jnp.int32),
            pltpu.VMEM((TPT // 32, 32), jnp.int32),
            pltpu.VMEM((16,), jnp.int32),
            pltpu.VMEM_SHARED((NSUB * 16,), jnp.int32),
            pltpu.VMEM((NSUB * 16,), jnp.int32),
            pltpu.VMEM((32,), jnp.int32),
            pltpu.VMEM((32,), jnp.int32),
            pltpu.VMEM((32, H), jnp.float32),
            pltpu.VMEM((32, H), jnp.float32),
            pltpu.SemaphoreType.DMA,
            pltpu.SemaphoreType.DMA,
            pltpu.SemaphoreType.DMA,
        ],
    )
    return f(gate, xf)


def _glu_body(meta_ref, x_ref, w1_ref, v1_ref, w2_ref, ws_ref, out_ref):
    t = pl.program_id(0)

    @pl.when(t < meta_ref[NT])
    def _():
        x = x_ref[...].astype(jnp.bfloat16)
        w1 = w1_ref[0].astype(jnp.bfloat16)
        v1 = v1_ref[0].astype(jnp.bfloat16)
        w2 = w2_ref[0].astype(jnp.bfloat16)
        gp = lax.dot_general(
            x, w1, (((1,), (1,)), ((), ())), preferred_element_type=jnp.float32
        )
        up = lax.dot_general(
            x, v1, (((1,), (1,)), ((), ())), preferred_element_type=jnp.float32
        )
        inter = (gp * lax.logistic(gp)) * up
        dn = lax.dot_general(
            inter.astype(jnp.bfloat16), w2,
            (((1,), (0,)), ((), ())), preferred_element_type=jnp.float32
        )
        out_ref[...] = dn * ws_ref[0]


def _glu_grouped(xs, w1b, v1b, w2b, ws3d, meta):
    grid_spec = pltpu.PrefetchScalarGridSpec(
        num_scalar_prefetch=1,
        grid=(NT,),
        in_specs=[
            pl.BlockSpec((BM, H), lambda t, m: (jnp.minimum(t, m[NT] - 1), 0)),
            pl.BlockSpec((1, F, H), lambda t, m: (m[t], 0, 0)),
            pl.BlockSpec((1, F, H), lambda t, m: (m[t], 0, 0)),
            pl.BlockSpec((1, F, H), lambda t, m: (m[t], 0, 0)),
            pl.BlockSpec((1, BM, 1), lambda t, m: (t, 0, 0)),
        ],
        out_specs=pl.BlockSpec((BM, H), lambda t, m: (t, 0)),
    )
    return pl.pallas_call(
        _glu_body,
        grid_spec=grid_spec,
        out_shape=jax.ShapeDtypeStruct((ROWS, H), jnp.float32),
        compiler_params=pltpu.CompilerParams(vmem_limit_bytes=58 * 1024 * 1024),
    )(meta, xs, w1b, v1b, w2b, ws3d)


def _combine_body(xo_hbm, pos0_hbm, pos1_hbm, out_hbm,
                  p0_v, p1_v, buf0, buf1, obuf, sem):
    c = lax.axis_index("c")
    s = lax.axis_index("s")
    wid = s * 2 + c
    t0 = wid * TPW

    for ch in range(TPW // 16):
        pltpu.sync_copy(pos0_hbm.at[pl.ds(t0 + ch * 16, 16)], p0_v.at[ch])
        pltpu.sync_copy(pos1_hbm.at[pl.ds(t0 + ch * 16, 16)], p1_v.at[ch])
        pltpu.async_copy(xo_hbm.at[p0_v.at[ch]], buf0, sem).wait()
        pltpu.async_copy(xo_hbm.at[p1_v.at[ch]], buf1, sem).wait()

        def _row(r, _):
            for col in range(H // 16):
                sl = pl.ds(col * 16, 16)
                obuf[r, sl] = buf0[r, sl] + buf1[r, sl]
            return _

        lax.fori_loop(0, 16, _row, 0)
        pltpu.sync_copy(obuf, out_hbm.at[pl.ds(t0 + ch * 16, 16)])


def _combine(xo, pos0, pos1):
    f = pl.kernel(
        _combine_body,
        out_type=jax.ShapeDtypeStruct((T, H), jnp.float32),
        mesh=_MESH,
        scratch_types=[
            pltpu.VMEM((TPW // 16, 16), jnp.int32),
            pltpu.VMEM((TPW // 16, 16), jnp.int32),
            pltpu.VMEM((16, H), jnp.float32),
            pltpu.VMEM((16, H), jnp.float32),
            pltpu.VMEM((16, H), jnp.float32),
            pltpu.SemaphoreType.DMA,
        ],
    )
    return f(xo, pos0, pos1)


def kernel(x, W_router, w1, v1, w2):
    bsz, q_len, hidden = x.shape
    xf = x.reshape(-1, hidden)
    gate = _router(xf, W_router)
    meta, pos0, pos1, ws, xs = _sort_scatter(gate, xf)
    xo = _glu_grouped(xs, w1, v1, w2, ws.reshape(NT, BM, 1), meta)
    out = _combine(xo, pos0.reshape(T), pos1.reshape(T))
    return out.reshape(bsz, q_len, hidden)


# speedup vs baseline: 1.2646x; 1.0160x over previous
import functools

import jax
import jax.numpy as jnp
from jax import lax
from jax.experimental import pallas as pl
from jax.experimental.pallas import tpu as pltpu
from jax.experimental.pallas import tpu_sc as plsc

H = 1024
F = 2048
E = 8
T = 2048

BM = 256
NT = 24
ROWS = NT * BM
NSUB = 16
TPT = T // NSUB
TPW = T // 32

_MESH = plsc.VectorSubcoreMesh(core_axis_name="c", subcore_axis_name="s")


def _router_body(x_ref, wr_ref, gate_ref):
    x = x_ref[...]
    wr = wr_ref[...]
    logits = lax.dot_general(
        x, wr, (((1,), (1,)), ((), ())), preferred_element_type=jnp.float32
    )
    m = jnp.max(logits, axis=-1, keepdims=True)
    p = jnp.exp(logits - m)
    w = p / jnp.sum(p, axis=-1, keepdims=True)
    eidx = lax.broadcasted_iota(jnp.int32, w.shape, 1)
    w1 = jnp.max(w, axis=-1, keepdims=True)
    first1 = jnp.min(jnp.where(w == w1, eidx, E), axis=-1, keepdims=True)
    m1 = eidx == first1
    wm = jnp.where(m1, -jnp.inf, w)
    w2 = jnp.max(wm, axis=-1, keepdims=True)
    first2 = jnp.min(jnp.where(wm == w2, eidx, E), axis=-1, keepdims=True)
    m2 = eidx == first2
    s = w1 + w2
    gate_ref[...] = jnp.where(m1, w1 / s, jnp.where(m2, w2 / s, 0.0))


def _router(xf, W_router):
    BMR = 1024
    return pl.pallas_call(
        _router_body,
        grid=(T // BMR,),
        in_specs=[
            pl.BlockSpec((BMR, H), lambda m: (m, 0)),
            pl.BlockSpec((E, H), lambda m: (0, 0)),
        ],
        out_specs=pl.BlockSpec((BMR, E), lambda m: (m, 0)),
        out_shape=jax.ShapeDtypeStruct((T, E), jnp.float32),
    )(xf, W_router)


def _splat(x):
    return jnp.full((16,), x, jnp.int32)


def _scan16(x, buf):
    for sh in (1, 2, 4, 8):
        buf[pl.ds(16, 16)] = x
        x = x + buf[pl.ds(16 - sh, 16)]
    return x


def _sort_body(gate_hbm, x_hbm, meta_hbm, pos0_hbm, pos1_hbm, ws_hbm,
               xs_hbm, gate_v, e0_v, e1_v, w0_v, w1_v, d0_v, d1_v,
               cntrow_v, cnt_sh, cnt_v, meta_v, scan_v, xrow_v, xrow2_v,
               sem, sem2, wsem):
    c = lax.axis_index("c")
    s = lax.axis_index("s")
    t0 = s * TPT
    lane = lax.broadcasted_iota(jnp.int32, (16,), 0)
    zv = jnp.zeros((16,), jnp.int32)
    ones = zv + 1
    scan_v[pl.ds(0, 16)] = zv

    pltpu.sync_copy(gate_hbm.at[:, pl.ds(t0, TPT)], gate_v)
    cnt = [zv for _ in range(E)]
    for g in range(TPT // 16):
        e0 = _splat(0)
        e1 = _splat(0)
        w0 = jnp.zeros((16,), jnp.float32)
        w1 = jnp.zeros((16,), jnp.float32)
        nseen = zv
        for e in range(E):
            ge = gate_v[e, pl.ds(g * 16, 16)]
            nz = ge > jnp.zeros((16,), jnp.float32)
            take0 = nz & (nseen == _splat(0))
            take1 = nz & (nseen == _splat(1))
            e0 = jnp.where(take0, _splat(e), e0)
            w0 = jnp.where(take0, ge, w0)
            e1 = jnp.where(take1, _splat(e), e1)
            w1 = jnp.where(take1, ge, w1)
            nseen = nseen + jnp.where(nz, ones, zv)
            cnt[e] = cnt[e] + jnp.where(take0, ones, zv) + jnp.where(take1, ones, zv)
        e0_v[pl.ds(g * 16, 16)] = e0
        e1_v[pl.ds(g * 16, 16)] = e1
        w0_v[pl.ds(g * 16, 16)] = w0
        w1_v[pl.ds(g * 16, 16)] = w1

    crow = zv
    for e in range(E):
        tot = _scan16(cnt[e], scan_v)[15]
        crow = jnp.where(lane == _splat(e), zv + tot, crow)
    cntrow_v[...] = crow
    pltpu.sync_copy(cntrow_v, cnt_sh.at[pl.ds(s * 16, 16)])
    plsc.subcore_barrier()

    pltpu.sync_copy(cnt_sh, cnt_v)
    rows = [cnt_v[pl.ds(t * 16, 16)] for t in range(NSUB)]
    svec = zv + s
    total_vec = rows[0]
    prefix_vec = rows[0] * jnp.where(_splat(0) < svec, ones, zv)
    for t in range(1, NSUB):
        total_vec = total_vec + rows[t]
        prefix_vec = prefix_vec + rows[t] * jnp.where(_splat(t) < svec, ones, zv)
    base = []
    cb = [jnp.int32(0)]
    for e in range(E):
        total = total_vec[e]
        nblk = (total + BM - 1) // BM
        base.append(cb[e] * BM + prefix_vec[e])
        cb.append(cb[e] + nblk)
    n_active = cb[E]

    run = [jnp.int32(0) for _ in range(E)]
    for g in range(TPT // 16):
        e0 = e0_v[pl.ds(g * 16, 16)]
        e1 = e1_v[pl.ds(g * 16, 16)]
        d0 = zv
        d1 = zv
        for e in range(E):
            m0 = e0 == _splat(e)
            mi0 = jnp.where(m0, ones, zv)
            incl0 = _scan16(mi0, scan_v)
            d0 = d0 + mi0 * ((zv + (base[e] + run[e])) + (incl0 - mi0))
            run[e] = run[e] + incl0[15]
            m1 = e1 == _splat(e)
            mi1 = jnp.where(m1, ones, zv)
            incl1 = _scan16(mi1, scan_v)
            d1 = d1 + mi1 * ((zv + (base[e] + run[e])) + (incl1 - mi1))
            run[e] = run[e] + incl1[15]
        d0_v[g // 2, pl.ds((g % 2) * 16, 16)] = d0
        d1_v[g // 2, pl.ds((g % 2) * 16, 16)] = d1

    @pl.when(c == 0)
    def _():
        pltpu.sync_copy(d0_v, pos0_hbm.at[s])
        pltpu.sync_copy(d1_v, pos1_hbm.at[s])

    @pl.when((c == 0) & (s == 0))
    def _():
        nav = zv + n_active
        lastev = zv
        ev0 = zv
        ev1 = zv
        for e in range(E):
            cbv = zv + cb[e + 1]
            lastev = lastev + jnp.where(nav - 1 >= cbv, ones, zv)
            ev0 = ev0 + jnp.where(lane >= cbv, ones, zv)
            ev1 = ev1 + jnp.where(lane + 16 >= cbv, ones, zv)
        ev0 = jnp.minimum(ev0, lastev)
        ev1 = jnp.minimum(ev1, lastev)
        ind = jnp.where(lane == _splat(NT - 16), ones, zv)
        ev1 = ev1 + ind * (nav - ev1)
        meta_v[pl.ds(0, 16)] = ev0
        meta_v[pl.ds(16, 16)] = ev1
        pltpu.sync_copy(meta_v, meta_hbm)

    def _scatter(d_v, w_v):
        nch = TPT // 32
        bufs = (xrow_v, xrow2_v)
        loads = [None] * nch
        stores = [None] * nch
        loads[0] = pltpu.async_copy(x_hbm.at[pl.ds(t0, 32)], bufs[0], sem)
        loads[1] = pltpu.async_copy(x_hbm.at[pl.ds(t0 + 32, 32)], bufs[1], sem2)
        wcopies = []
        for ch in range(nch):
            b = ch % 2
            loads[ch].wait()
            stores[ch] = pltpu.async_copy(bufs[b], xs_hbm.at[d_v.at[ch]],
                                          sem if b == 0 else sem2)
            wcopies.append(pltpu.async_copy(w_v.at[pl.ds(ch * 32, 32)],
                                            ws_hbm.at[d_v.at[ch]], wsem))
            if ch + 2 < nch:
                stores[ch].wait()
                loads[ch + 2] = pltpu.async_copy(
                    x_hbm.at[pl.ds(t0 + (ch + 2) * 32, 32)], bufs[b],
                    sem if b == 0 else sem2)
        stores[nch - 2].wait()
        stores[nch - 1].wait()
        for h in wcopies:
            h.wait()

    @pl.when(c == 0)
    def _():
        _scatter(d0_v, w0_v)

    @pl.when(c == 1)
    def _():
        _scatter(d1_v, w1_v)


def _sort_scatter(gate, xf):
    f = pl.kernel(
        _sort_body,
        out_type=[
            jax.ShapeDtypeStruct((32,), jnp.int32),
            jax.ShapeDtypeStruct((NSUB, TPT // 32, 32), jnp.int32),
            jax.ShapeDtypeStruct((NSUB, TPT // 32, 32), jnp.int32),
            jax.ShapeDtypeStruct((ROWS,), jnp.float32),
            jax.ShapeDtypeStruct((ROWS, H), jnp.float32),
        ],
        mesh=_MESH,
        scratch_types=[
            pltpu.VMEM((E, TPT), jnp.float32),
            pltpu.VMEM((TPT,), jnp.int32),
            pltpu.VMEM((TPT,), jnp.int32),
            pltpu.VMEM((TPT,), jnp.float32),
            pltpu.VMEM((TPT,), jnp.float32),
            pltpu.VMEM((TPT // 32, 32), jnp.int32),
            pltpu.VMEM((TPT // 32, 32), jnp.int32),
            pltpu.VMEM((16,), jnp.int32),
            pltpu.VMEM_SHARED((NSUB * 16,), jnp.int32),
            pltpu.VMEM((NSUB * 16,), jnp.int32),
            pltpu.VMEM((32,), jnp.int32),
            pltpu.VMEM((32,), jnp.int32),
            pltpu.VMEM((32, H), jnp.float32),
            pltpu.VMEM((32, H), jnp.float32),
            pltpu.SemaphoreType.DMA,
            pltpu.SemaphoreType.DMA,
            pltpu.SemaphoreType.DMA,
        ],
    )
    return f(gate, xf)


def _glu_body(meta_ref, x_ref, w1_ref, v1_ref, w2_ref, ws_ref, out_ref):
    t = pl.program_id(0)

    @pl.when(t < meta_ref[NT])
    def _():
        x = x_ref[...].astype(jnp.bfloat16)
        w1 = w1_ref[0].astype(jnp.bfloat16)
        v1 = v1_ref[0].astype(jnp.bfloat16)
        w2 = w2_ref[0].astype(jnp.bfloat16)
        gp = lax.dot_general(
            x, w1, (((1,), (1,)), ((), ())), preferred_element_type=jnp.float32
        )
        up = lax.dot_general(
            x, v1, (((1,), (1,)), ((), ())), preferred_element_type=jnp.float32
        )
        inter = (gp * lax.logistic(gp)) * up
        dn = lax.dot_general(
            inter.astype(jnp.bfloat16), w2,
            (((1,), (0,)), ((), ())), preferred_element_type=jnp.float32
        )
        out_ref[...] = dn * ws_ref[0]


def _glu_grouped(xs, w1b, v1b, w2b, ws3d, meta):
    grid_spec = pltpu.PrefetchScalarGridSpec(
        num_scalar_prefetch=1,
        grid=(NT,),
        in_specs=[
            pl.BlockSpec((BM, H), lambda t, m: (jnp.minimum(t, m[NT] - 1), 0)),
            pl.BlockSpec((1, F, H), lambda t, m: (m[t], 0, 0)),
            pl.BlockSpec((1, F, H), lambda t, m: (m[t], 0, 0)),
            pl.BlockSpec((1, F, H), lambda t, m: (m[t], 0, 0)),
            pl.BlockSpec((1, BM, 1), lambda t, m: (t, 0, 0)),
        ],
        out_specs=pl.BlockSpec((BM, H), lambda t, m: (t, 0)),
    )
    return pl.pallas_call(
        _glu_body,
        grid_spec=grid_spec,
        out_shape=jax.ShapeDtypeStruct((ROWS, H), jnp.float32),
        compiler_params=pltpu.CompilerParams(vmem_limit_bytes=58 * 1024 * 1024),
    )(meta, xs, w1b, v1b, w2b, ws3d)


def _combine_body(xo_hbm, pos0_hbm, pos1_hbm, out_hbm,
                  p0_v, p1_v, buf0, buf1, obuf, sem):
    c = lax.axis_index("c")
    s = lax.axis_index("s")
    wid = s * 2 + c
    t0 = wid * TPW

    for ch in range(TPW // 16):
        pltpu.sync_copy(pos0_hbm.at[pl.ds(t0 + ch * 16, 16)], p0_v.at[ch])
        pltpu.sync_copy(pos1_hbm.at[pl.ds(t0 + ch * 16, 16)], p1_v.at[ch])
        pltpu.async_copy(xo_hbm.at[p0_v.at[ch]], buf0, sem).wait()
        pltpu.async_copy(xo_hbm.at[p1_v.at[ch]], buf1, sem).wait()

        def _row(r, _):
            for col in range(H // 16):
                sl = pl.ds(col * 16, 16)
                obuf[r, sl] = buf0[r, sl] + buf1[r, sl]
            return _

        lax.fori_loop(0, 16, _row, 0)
        pltpu.sync_copy(obuf, out_hbm.at[pl.ds(t0 + ch * 16, 16)])


def _combine(xo, pos0, pos1):
    f = pl.kernel(
        _combine_body,
        out_type=jax.ShapeDtypeStruct((T, H), jnp.float32),
        mesh=_MESH,
        scratch_types=[
            pltpu.VMEM((TPW // 16, 16), jnp.int32),
            pltpu.VMEM((TPW // 16, 16), jnp.int32),
            pltpu.VMEM((16, H), jnp.float32),
            pltpu.VMEM((16, H), jnp.float32),
            pltpu.VMEM((16, H), jnp.float32),
            pltpu.SemaphoreType.DMA,
        ],
    )
    return f(xo, pos0, pos1)


def kernel(x, W_router, w1, v1, w2):
    bsz, q_len, hidden = x.shape
    xf = x.reshape(-1, hidden)
    gate = _router(xf, W_router)
    meta, pos0, pos1, ws, xs = _sort_scatter(gate.T, xf)
    xo = _glu_grouped(xs, w1, v1, w2, ws.reshape(NT, BM, 1), meta)
    out = _combine(xo, pos0.reshape(T), pos1.reshape(T))
    return out.reshape(bsz, q_len, hidden)
